# Initial kernel scaffold; baseline (speedup 1.0000x reference)
import jax
import jax.numpy as jnp
from jax.experimental import pallas as pl


def kernel(x, embeddings):
    raise NotImplementedError("write your pallas kernel here")


# speedup vs baseline: 28.8704x; 28.8704x over previous
import jax
import jax.numpy as jnp
from jax import lax
from jax.experimental import pallas as pl
from jax.experimental.pallas import tpu as pltpu
from jax.experimental.pallas import tpu_sc as plsc

N_LEVELS = 16
N_FEATURES = 2
LOG2_HASH = 19
HASH_SIZE = 1 << LOG2_HASH
HMASK = HASH_SIZE - 1
BASE_RES = 16
B_PTS = 1048576

_RES = [BASE_RES << l for l in range(N_LEVELS)]

NC = 2
NS = 16
NW = NC * NS
LANES = 16

PT_PER_TILE = B_PTS // NW
ROW_W = 8
LOG2_ROW_W = ROW_W.bit_length() - 1
RSH = 5 - LOG2_ROW_W
CHUNK = 2048 // ROW_W
N_CHUNKS = PT_PER_TILE // CHUNK
X_STAGE = 8192
CHUNKS_PER_STAGE = X_STAGE // CHUNK
GC = 128

N_SMALL = 11
BIG = list(range(N_SMALL, N_LEVELS))
N_BIG = len(BIG)

_R = [_RES[l] + 2 for l in range(N_SMALL)]
_OFF = []
_o = 0
for _l in range(N_SMALL):
    _OFF.append(_o)
    _o += -(-(2 * _R[_l]) // 8) * 8
TAB_WORDS = _o + 512

OUT_HALF = CHUNK * 2 * N_LEVELS
IDX_WORDS = N_BIG * 2 * CHUNK
ROW_CAP = N_BIG * 2 * CHUNK


def _body(x_hbm, embr, out_hbm, tab, xbuf, bigidx, rows, outb, sem_g, sem_o):
    wid = lax.axis_index("s") * NC + lax.axis_index("c")
    tile_base = wid * PT_PER_TILE

    iota = lax.iota(jnp.int32, LANES)
    lane32 = iota * (2 * N_LEVELS)
    half = iota >> 1
    parity = iota & 1
    c0 = jnp.zeros((LANES,), jnp.int32)
    c1 = jnp.ones((LANES,), jnp.int32)

    for l in range(N_SMALL):
        rmax = _R[l] - 1
        for rs in range(0, _R[l], ROW_CAP):
            n = min(ROW_CAP, _R[l] - rs)
            n_pad = -(-n // GC) * GC

            def _stage_idx(k, _, l=l, rs=rs, rmax=rmax):
                j = jnp.minimum(iota + (rs + k * LANES), rmax)
                bigidx[pl.ds(k * LANES, LANES)] = (
                    (j << RSH) + ((2 * l) >> LOG2_ROW_W))
                return 0

            lax.fori_loop(0, n_pad // LANES, _stage_idx, 0)
            for cc in range(0, n_pad, GC):
                pltpu.async_copy(
                    embr.at[bigidx.at[pl.ds(cc, GC)]],
                    rows.at[pl.ds(cc, GC)],
                    sem_g,
                )
            for cc in range(0, n_pad, GC):
                pltpu.make_async_copy(
                    embr.at[bigidx.at[pl.ds(cc, GC)]],
                    rows.at[pl.ds(cc, GC)],
                    sem_g,
                ).wait()

            def _extract(m, _, l=l, rs=rs):
                v = plsc.load_gather(
                    rows, [half + m * 8, parity + ((2 * l) % ROW_W)])
                tab[pl.ds(_OFF[l] + 2 * rs + m * LANES, LANES)] = v
                return 0

            lax.fori_loop(0, (2 * n_pad) // LANES, _extract, 0)

    def _chunk(g, _):
        pt_base = tile_base + g * CHUNK
        xoff = lax.rem(g, CHUNKS_PER_STAGE) * CHUNK
        ob = lax.rem(g, 2) * OUT_HALF

        @pl.when(lax.rem(g, CHUNKS_PER_STAGE) == 0)
        def _():
            pltpu.sync_copy(
                x_hbm.at[pl.ds(tile_base + (g // CHUNKS_PER_STAGE) * X_STAGE,
                               X_STAGE)],
                xbuf,
            )

        @pl.when(g >= 2)
        def _():
            pltpu.make_async_copy(
                outb.at[pl.ds(ob, OUT_HALF)],
                out_hbm.at[pl.ds((pt_base - 2 * CHUNK) * 2 * N_LEVELS,
                                 OUT_HALF)],
                sem_o,
            ).wait()

        def _p1a(k, _):
            x16 = xbuf[pl.ds(xoff + k * LANES, LANES)]
            x16 = jnp.minimum(jnp.maximum(x16, 0.0), 1.0)
            for l in BIG:
                bl = l - N_SMALL
                pos = x16 * float(_RES[l])
                i0 = pos.astype(jnp.int32)
                h0 = i0
                h1 = i0 + 1
                if l == N_LEVELS - 1:
                    h0 = h0 & HMASK
                    h1 = h1 & HMASK
                rsub = (2 * l) >> LOG2_ROW_W
                bigidx[pl.ds(bl * 2 * CHUNK + k * LANES, LANES)] = (
                    (h0 << RSH) + rsub)
                bigidx[pl.ds(bl * 2 * CHUNK + CHUNK + k * LANES, LANES)] = (
                    (h1 << RSH) + rsub)
            return 0

        lax.fori_loop(0, CHUNK // LANES, _p1a, 0)

        for bl in range(N_BIG):
            for cc in range(0, 2 * CHUNK, GC):
                pltpu.async_copy(
                    embr.at[bigidx.at[pl.ds(bl * 2 * CHUNK + cc, GC)]],
                    rows.at[pl.ds(bl * 2 * CHUNK + cc, GC)],
                    sem_g,
                )

        def _p1b(k, _):
            x16 = xbuf[pl.ds(xoff + k * LANES, LANES)]
            x16 = jnp.minimum(jnp.maximum(x16, 0.0), 1.0)
            rb32 = lane32 + (ob + k * (LANES * 2 * N_LEVELS))
            for l in range(N_SMALL):
                pos = x16 * float(_RES[l])
                i0 = pos.astype(jnp.int32)
                w = pos - i0.astype(jnp.float32)
                a = (i0 << 1) + _OFF[l]
                e0x = plsc.load_gather(tab, [a])
                e0y = plsc.load_gather(tab, [a + 1])
                e1x = plsc.load_gather(tab, [a + 2])
                e1y = plsc.load_gather(tab, [a + 3])
                ox = e0x + w * (e1x - e0x)
                oy = e0y + w * (e1y - e0y)
                plsc.store_scatter(outb, [rb32 + 2 * l], ox)
                plsc.store_scatter(outb, [rb32 + (2 * l + 1)], oy)
            return 0

        lax.fori_loop(0, CHUNK // LANES, _p1b, 0)

        for bl in range(N_BIG):
            for cc in range(0, 2 * CHUNK, GC):
                pltpu.make_async_copy(
                    embr.at[bigidx.at[pl.ds(bl * 2 * CHUNK + cc, GC)]],
                    rows.at[pl.ds(bl * 2 * CHUNK + cc, GC)],
                    sem_g,
                ).wait()

        def _p2(m, _):
            x16 = xbuf[pl.ds(xoff + m * LANES, LANES)]
            x16 = jnp.minimum(jnp.maximum(x16, 0.0), 1.0)
            rb32 = lane32 + (ob + m * (LANES * 2 * N_LEVELS))
            for l in BIG:
                bl = l - N_SMALL
                pos = x16 * float(_RES[l])
                i0 = pos.astype(jnp.int32)
                w = pos - i0.astype(jnp.float32)
                rA = iota + (bl * 2 * CHUNK + m * LANES)
                rB = rA + CHUNK
                cx = c0 + ((2 * l) % ROW_W)
                e0x = plsc.load_gather(rows, [rA, cx])
                e0y = plsc.load_gather(rows, [rA, cx + 1])
                e1x = plsc.load_gather(rows, [rB, cx])
                e1y = plsc.load_gather(rows, [rB, cx + 1])
                ox = e0x + w * (e1x - e0x)
                oy = e0y + w * (e1y - e0y)
                plsc.store_scatter(outb, [rb32 + 2 * l], ox)
                plsc.store_scatter(outb, [rb32 + (2 * l + 1)], oy)
            return 0

        lax.fori_loop(0, CHUNK // LANES, _p2, 0)

        pltpu.async_copy(
            outb.at[pl.ds(ob, OUT_HALF)],
            out_hbm.at[pl.ds(pt_base * 2 * N_LEVELS, OUT_HALF)],
            sem_o,
        )
        return 0

    lax.fori_loop(0, N_CHUNKS, _chunk, 0)

    for gg in (N_CHUNKS - 2, N_CHUNKS - 1):
        pltpu.make_async_copy(
            outb.at[pl.ds((gg % 2) * OUT_HALF, OUT_HALF)],
            out_hbm.at[pl.ds((tile_base + gg * CHUNK) * 2 * N_LEVELS,
                             OUT_HALF)],
            sem_o,
        ).wait()


@jax.jit
def kernel(x, embeddings):
    assert x.shape == (B_PTS,) and embeddings.shape == (HASH_SIZE,
                                                        2 * N_LEVELS)
    embr = embeddings.reshape(HASH_SIZE * 2 * N_LEVELS // ROW_W, ROW_W)
    mesh = plsc.VectorSubcoreMesh(core_axis_name="c", subcore_axis_name="s")
    out = pl.kernel(
        _body,
        out_type=jax.ShapeDtypeStruct((B_PTS * 2 * N_LEVELS,), jnp.float32),
        mesh=mesh,
        compiler_params=pltpu.CompilerParams(
            use_tc_tiling_on_sc=False, needs_layout_passes=False),
        scratch_types=[
            pltpu.VMEM((TAB_WORDS,), jnp.float32),
            pltpu.VMEM((X_STAGE,), jnp.float32),
            pltpu.VMEM((IDX_WORDS,), jnp.int32),
            pltpu.VMEM((ROW_CAP, ROW_W), jnp.float32),
            pltpu.VMEM((2 * OUT_HALF,), jnp.float32),
            pltpu.SemaphoreType.DMA,
            pltpu.SemaphoreType.DMA,
        ],
    )(x, embr)
    return out.reshape(B_PTS, 2 * N_LEVELS)
